# baseline (speedup 1.0000x reference)
import functools

import jax
import jax.numpy as jnp
from jax import lax
from jax.experimental import pallas as pl
from jax.experimental.pallas import tpu as pltpu
from jax.experimental.pallas import tpu_sc as plsc

B, T, D = 16, 1024, 256
L = 8192
NC, NS = 2, 16
NW = NC * NS
PW = B * L // NW
CHUNK = 128
NCHUNK = PW // CHUNK
SROWS = CHUNK + 8
VL = 16
DV = D // VL


def _body(x_hbm, dur_hbm, z_hbm, out_hbm, mel_hbm,
          dur_v, a_v, idx_v, sbuf, ob0, ob1, mel_v,
          gsem, wsem):
    cid = lax.axis_index("c")
    sid = lax.axis_index("s")
    wid = sid * NC + cid
    b = wid // 2
    half = wid % 2
    p0 = half * (L // 2)
    lane = jnp.arange(VL, dtype=jnp.int32)

    pltpu.sync_copy(dur_hbm.at[b], dur_v.at[pl.ds(0, T)])
    dur_v[pl.ds(T, VL)] = jnp.zeros((VL,), jnp.int32)

    pltpu.sync_copy(z_hbm, a_v)

    def scat_step(j, carry, base):
        v = dur_v[pl.ds(j * VL, VL)]
        s = plsc.cumsum(v) + carry
        i_vec = lane + j * VL
        d_next = dur_v[pl.ds(j * VL + 1, VL)]
        run_end = (d_next != 0) | (i_vec == T - 1)
        local = s - p0
        m = run_end & (local >= 0) & (local < PW)
        plsc.store_scatter(a_v, (jnp.where(m, local, 0),), i_vec + 1, mask=m)
        base = base + jnp.sum(jnp.where(s < p0, 1, 0).astype(jnp.int32))
        return jnp.max(s), base

    def scat_body(jj, carry_base):
        carry, base = carry_base
        for k in range(2):
            carry, base = scat_step(jj * 2 + k, carry, base)
        return (carry, base)

    total, base = lax.fori_loop(
        0, T // VL // 2, scat_body, (jnp.int32(0), jnp.int32(0)))

    rowbase = b * T

    def idx_body(ii, carry):
        for k in range(4):
            i = ii * 4 + k
            v = a_v[pl.ds(i * VL, VL)]
            s = jnp.maximum(plsc.cummax(v), carry)
            idx_v[pl.ds(i * VL, VL)] = jnp.minimum(s, T - 1) + rowbase
            carry = jnp.max(s)
        return carry

    lax.fori_loop(0, PW // VL // 4, idx_body, base)
    idx_v[pl.ds(PW, VL)] = jnp.full((VL,), rowbase + T - 1, jnp.int32)

    @pl.when(half == 0)
    def _():
        mel_v[...] = jnp.full((VL,), total, jnp.int32)
        pltpu.sync_copy(mel_v, mel_hbm.at[b])

    n_valid = jnp.clip(total - p0, 0, PW)
    row0 = wid * PW

    def src_row(p):
        return jnp.min(idx_v[pl.ds(p, VL)])

    def drain_write():
        pltpu.make_async_copy(
            ob0.at[pl.ds(0, CHUNK * D)],
            out_hbm.at[pl.ds(row0 * D, CHUNK * D)], wsem).wait()

    def do_chunk(c, ob):
        c_lo = c * CHUNK
        r = jnp.clip(n_valid - c_lo, 0, CHUNK)
        lo_g = src_row(c_lo)
        hi_g = src_row(c_lo + jnp.maximum(r - 1, 0))
        span = hi_g - lo_g + 1
        start = pl.multiple_of(
            jnp.minimum((lo_g // 8) * 8, B * T - SROWS), 8)

        @pl.when((r > 0) & (span <= CHUNK))
        def _():
            pltpu.async_copy(
                x_hbm.at[pl.ds(start * D, SROWS * D)], sbuf, gsem).wait()

            def expand(pp, _):
                for k in range(2):
                    p = pp * 2 + k
                    so = jnp.clip(src_row(c_lo + p) - start,
                                  0, SROWS - 1) * D
                    po = p * D
                    for d in range(DV):
                        ob[pl.ds(po + d * VL, VL)] = \
                            sbuf[pl.ds(so + d * VL, VL)]
                return 0
            lax.fori_loop(0, (r + 1) // 2, expand, 0)

        @pl.when((r > 0) & (span > CHUNK))
        def _():
            def row_copy(p, _):
                g = src_row(c_lo + p)
                pltpu.sync_copy(x_hbm.at[pl.ds(g * D, D)],
                                ob.at[pl.ds(p * D, D)])
                return 0
            lax.fori_loop(0, r, row_copy, 0)

        def zrow(i, _):
            for k in range(4):
                po = (r + i * 4 + k) * D
                for d in range(DV):
                    ob[pl.ds(po + d * VL, VL)] = jnp.zeros((VL,), jnp.float32)
            return 0
        lax.fori_loop(0, (CHUNK - r + 3) // 4, zrow, 0)

        pltpu.async_copy(
            ob.at[pl.ds(0, CHUNK * D)],
            out_hbm.at[pl.ds((row0 + c_lo) * D, CHUNK * D)], wsem)

    def pair_body(cc, _):
        @pl.when(cc > 0)
        def _():
            drain_write()
            drain_write()
        do_chunk(cc * 2, ob0)
        do_chunk(cc * 2 + 1, ob1)
        return 0

    lax.fori_loop(0, NCHUNK // 2, pair_body, 0)
    drain_write()
    drain_write()


@functools.cache
def _regulate():
    return pl.kernel(
        _body,
        out_type=[
            jax.ShapeDtypeStruct((B * L * D,), jnp.float32),
            jax.ShapeDtypeStruct((B, VL), jnp.int32),
        ],
        name="length_regulator",
        mesh=plsc.VectorSubcoreMesh(core_axis_name="c", subcore_axis_name="s",
                                    num_cores=NC, num_subcores=NS),
        compiler_params=pltpu.CompilerParams(needs_layout_passes=False),
        scratch_types=[
            pltpu.VMEM((T + VL,), jnp.int32),
            pltpu.VMEM((PW,), jnp.int32),
            pltpu.VMEM((PW + VL,), jnp.int32),
            pltpu.VMEM((SROWS * D,), jnp.float32),
            pltpu.VMEM(((CHUNK + 4) * D,), jnp.float32),
            pltpu.VMEM(((CHUNK + 4) * D,), jnp.float32),
            pltpu.VMEM((VL,), jnp.int32),
            pltpu.SemaphoreType.DMA,
            pltpu.SemaphoreType.DMA,
        ],
    )


def kernel(x, duration, max_len):
    out_flat, mel2 = _regulate()(x.reshape(B * T * D),
                                 duration.astype(jnp.int32),
                                 jnp.zeros((PW,), jnp.int32))
    return out_flat.reshape(B, L, D), mel2[:, 0]


# speedup vs baseline: 1.7146x; 1.7146x over previous
import functools

import jax
import jax.numpy as jnp
from jax import lax
from jax.experimental import pallas as pl
from jax.experimental.pallas import tpu as pltpu
from jax.experimental.pallas import tpu_sc as plsc

B, T, D = 16, 1024, 256
L = 8192
NC, NS = 2, 16
NW = NC * NS
PW = B * L // NW
CHUNK = 128
NCHUNK = PW // CHUNK
SROWS = CHUNK + 8
VL = 16
DV = D // VL


def _body(x_hbm, dur_hbm, z_hbm, out_hbm, mel_hbm,
          dur_v, a_v, idx_v, sbuf, ob0, ob1, mel_v,
          gsem, wsem):
    cid = lax.axis_index("c")
    sid = lax.axis_index("s")
    wid = sid * NC + cid
    b = wid // 2
    half = wid % 2
    p0 = half * (L // 2)
    lane = jnp.arange(VL, dtype=jnp.int32)

    pltpu.sync_copy(dur_hbm.at[b], dur_v.at[pl.ds(0, T)])
    dur_v[pl.ds(T, VL)] = jnp.zeros((VL,), jnp.int32)

    pltpu.sync_copy(z_hbm, a_v)

    def scat_step(j, carry, base):
        v = dur_v[pl.ds(j * VL, VL)]
        s = plsc.cumsum(v) + carry
        i_vec = lane + j * VL
        d_next = dur_v[pl.ds(j * VL + 1, VL)]
        run_end = (d_next != 0) | (i_vec == T - 1)
        local = s - p0
        m = run_end & (local >= 0) & (local < PW)
        plsc.store_scatter(a_v, (jnp.where(m, local, 0),), i_vec + 1, mask=m)
        base = base + jnp.sum(jnp.where(s < p0, 1, 0).astype(jnp.int32))
        return jnp.max(s), base

    def scat_body(jj, carry_base):
        carry, base = carry_base
        for k in range(2):
            carry, base = scat_step(jj * 2 + k, carry, base)
        return (carry, base)

    total, base = lax.fori_loop(
        0, T // VL // 2, scat_body, (jnp.int32(0), jnp.int32(0)))

    rowbase = b * T

    def idx_body(ii, carry):
        for k in range(4):
            i = ii * 4 + k
            v = a_v[pl.ds(i * VL, VL)]
            s = jnp.maximum(plsc.cummax(v), carry)
            idx_v[pl.ds(i * VL, VL)] = jnp.minimum(s, T - 1) + rowbase
            carry = jnp.max(s)
        return carry

    lax.fori_loop(0, PW // VL // 4, idx_body, base)
    idx_v[pl.ds(PW, VL)] = jnp.full((VL,), rowbase + T - 1, jnp.int32)

    @pl.when(half == 0)
    def _():
        mel_v[...] = jnp.full((VL,), total, jnp.int32)
        pltpu.sync_copy(mel_v, mel_hbm.at[b])

    n_valid = jnp.clip(total - p0, 0, PW)
    row0 = wid * PW

    def src_row(p):
        return jnp.min(idx_v[pl.ds(p, VL)])

    def drain_write():
        pltpu.make_async_copy(
            ob0.at[pl.ds(0, CHUNK * D)],
            out_hbm.at[pl.ds(row0 * D, CHUNK * D)], wsem).wait()

    def do_chunk(c, ob):
        c_lo = c * CHUNK
        r = jnp.clip(n_valid - c_lo, 0, CHUNK)
        lo_g = src_row(c_lo)
        hi_g = src_row(c_lo + jnp.maximum(r - 1, 0))
        span = hi_g - lo_g + 1
        start = pl.multiple_of(
            jnp.minimum((lo_g // 8) * 8, B * T - SROWS), 8)

        @pl.when((r > 0) & (span <= CHUNK))
        def _():
            pltpu.async_copy(
                x_hbm.at[pl.ds(start * D, SROWS * D)], sbuf, gsem).wait()

            @plsc.parallel_loop(0, r, step=1, unroll=4)
            def expand(p):
                so = jnp.clip(src_row(c_lo + p) - start, 0, SROWS - 1) * D
                po = p * D
                for d in range(DV):
                    ob[pl.ds(po + d * VL, VL)] = sbuf[pl.ds(so + d * VL, VL)]

        @pl.when((r > 0) & (span > CHUNK))
        def _():
            def row_copy(p, _):
                g = src_row(c_lo + p)
                pltpu.sync_copy(x_hbm.at[pl.ds(g * D, D)],
                                ob.at[pl.ds(p * D, D)])
                return 0
            lax.fori_loop(0, r, row_copy, 0)

        @plsc.parallel_loop(r, CHUNK, step=1, unroll=4)
        def zrow(p):
            po = p * D
            for d in range(DV):
                ob[pl.ds(po + d * VL, VL)] = jnp.zeros((VL,), jnp.float32)

        pltpu.async_copy(
            ob.at[pl.ds(0, CHUNK * D)],
            out_hbm.at[pl.ds((row0 + c_lo) * D, CHUNK * D)], wsem)

    def pair_body(cc, _):
        @pl.when(cc > 0)
        def _():
            drain_write()
            drain_write()
        do_chunk(cc * 2, ob0)
        do_chunk(cc * 2 + 1, ob1)
        return 0

    lax.fori_loop(0, NCHUNK // 2, pair_body, 0)
    drain_write()
    drain_write()


@functools.cache
def _regulate():
    return pl.kernel(
        _body,
        out_type=[
            jax.ShapeDtypeStruct((B * L * D,), jnp.float32),
            jax.ShapeDtypeStruct((B, VL), jnp.int32),
        ],
        name="length_regulator",
        mesh=plsc.VectorSubcoreMesh(core_axis_name="c", subcore_axis_name="s",
                                    num_cores=NC, num_subcores=NS),
        compiler_params=pltpu.CompilerParams(needs_layout_passes=False),
        scratch_types=[
            pltpu.VMEM((T + VL,), jnp.int32),
            pltpu.VMEM((PW,), jnp.int32),
            pltpu.VMEM((PW + VL,), jnp.int32),
            pltpu.VMEM((SROWS * D,), jnp.float32),
            pltpu.VMEM(((CHUNK + 4) * D,), jnp.float32),
            pltpu.VMEM(((CHUNK + 4) * D,), jnp.float32),
            pltpu.VMEM((VL,), jnp.int32),
            pltpu.SemaphoreType.DMA,
            pltpu.SemaphoreType.DMA,
        ],
    )


def kernel(x, duration, max_len):
    out_flat, mel2 = _regulate()(x.reshape(B * T * D),
                                 duration.astype(jnp.int32),
                                 jnp.zeros((PW,), jnp.int32))
    return out_flat.reshape(B, L, D), mel2[:, 0]


# speedup vs baseline: 1.7176x; 1.0018x over previous
import functools

import jax
import jax.numpy as jnp
from jax import lax
from jax.experimental import pallas as pl
from jax.experimental.pallas import tpu as pltpu
from jax.experimental.pallas import tpu_sc as plsc

B, T, D = 16, 1024, 256
L = 8192
NC, NS = 2, 16
NW = NC * NS
PW = B * L // NW
CHUNK = 128
NCHUNK = PW // CHUNK
SROWS = CHUNK + 8
VL = 16
DV = D // VL


def _body(x_hbm, dur_hbm, z_hbm, out_hbm, mel_hbm,
          dur_v, a_v, idx_v, sbuf, ob0, ob1, mel_v,
          gsem, wsem):
    cid = lax.axis_index("c")
    sid = lax.axis_index("s")
    wid = sid * NC + cid
    b = wid // 2
    half = wid % 2
    p0 = half * (L // 2)
    lane = jnp.arange(VL, dtype=jnp.int32)

    pltpu.sync_copy(dur_hbm.at[b], dur_v.at[pl.ds(0, T)])
    dur_v[pl.ds(T, VL)] = jnp.zeros((VL,), jnp.int32)

    pltpu.sync_copy(z_hbm, a_v)

    def scat_step(j, carry, base):
        v = dur_v[pl.ds(j * VL, VL)]
        s = plsc.cumsum(v) + carry
        i_vec = lane + j * VL
        d_next = dur_v[pl.ds(j * VL + 1, VL)]
        run_end = (d_next != 0) | (i_vec == T - 1)
        local = s - p0
        m = run_end & (local >= 0) & (local < PW)
        plsc.store_scatter(a_v, (jnp.where(m, local, 0),), i_vec + 1, mask=m)
        base = base + jnp.sum(jnp.where(s < p0, 1, 0).astype(jnp.int32))
        return jnp.max(s), base

    @plsc.parallel_loop(0, T // VL, step=1, unroll=4,
                        carry=(jnp.int32(0), jnp.int32(0)))
    def _scat(j, carry_base):
        carry, base = carry_base
        return scat_step(j, carry, base)

    total, base = _scat

    rowbase = b * T

    @plsc.parallel_loop(0, PW // VL, step=1, unroll=4, carry=base)
    def _sweep(i, carry):
        v = a_v[pl.ds(i * VL, VL)]
        s = jnp.maximum(plsc.cummax(v), carry)
        idx_v[pl.ds(i * VL, VL)] = jnp.minimum(s, T - 1) + rowbase
        return jnp.max(s)
    idx_v[pl.ds(PW, VL)] = jnp.full((VL,), rowbase + T - 1, jnp.int32)

    @pl.when(half == 0)
    def _():
        mel_v[...] = jnp.full((VL,), total, jnp.int32)
        pltpu.sync_copy(mel_v, mel_hbm.at[b])

    n_valid = jnp.clip(total - p0, 0, PW)
    row0 = wid * PW

    def src_row(p):
        return jnp.min(idx_v[pl.ds(p, VL)])

    def drain_write():
        pltpu.make_async_copy(
            ob0.at[pl.ds(0, CHUNK * D)],
            out_hbm.at[pl.ds(row0 * D, CHUNK * D)], wsem).wait()

    def do_chunk(c, ob):
        c_lo = c * CHUNK
        r = jnp.clip(n_valid - c_lo, 0, CHUNK)
        lo_g = src_row(c_lo)
        hi_g = src_row(c_lo + jnp.maximum(r - 1, 0))
        span = hi_g - lo_g + 1
        start = pl.multiple_of(
            jnp.minimum((lo_g // 8) * 8, B * T - SROWS), 8)

        @pl.when((r > 0) & (span <= CHUNK))
        def _():
            pltpu.async_copy(
                x_hbm.at[pl.ds(start * D, SROWS * D)], sbuf, gsem).wait()

            @plsc.parallel_loop(0, r, step=1, unroll=4)
            def expand(p):
                so = jnp.clip(src_row(c_lo + p) - start, 0, SROWS - 1) * D
                po = p * D
                for d in range(DV):
                    ob[pl.ds(po + d * VL, VL)] = sbuf[pl.ds(so + d * VL, VL)]

        @pl.when((r > 0) & (span > CHUNK))
        def _():
            def row_copy(p, _):
                g = src_row(c_lo + p)
                pltpu.sync_copy(x_hbm.at[pl.ds(g * D, D)],
                                ob.at[pl.ds(p * D, D)])
                return 0
            lax.fori_loop(0, r, row_copy, 0)

        @plsc.parallel_loop(r, CHUNK, step=1, unroll=4)
        def zrow(p):
            po = p * D
            for d in range(DV):
                ob[pl.ds(po + d * VL, VL)] = jnp.zeros((VL,), jnp.float32)

        pltpu.async_copy(
            ob.at[pl.ds(0, CHUNK * D)],
            out_hbm.at[pl.ds((row0 + c_lo) * D, CHUNK * D)], wsem)

    def pair_body(cc, _):
        @pl.when(cc > 0)
        def _():
            drain_write()
            drain_write()
        do_chunk(cc * 2, ob0)
        do_chunk(cc * 2 + 1, ob1)
        return 0

    lax.fori_loop(0, NCHUNK // 2, pair_body, 0)
    drain_write()
    drain_write()


@functools.cache
def _regulate():
    return pl.kernel(
        _body,
        out_type=[
            jax.ShapeDtypeStruct((B * L * D,), jnp.float32),
            jax.ShapeDtypeStruct((B, VL), jnp.int32),
        ],
        name="length_regulator",
        mesh=plsc.VectorSubcoreMesh(core_axis_name="c", subcore_axis_name="s",
                                    num_cores=NC, num_subcores=NS),
        compiler_params=pltpu.CompilerParams(needs_layout_passes=False),
        scratch_types=[
            pltpu.VMEM((T + VL,), jnp.int32),
            pltpu.VMEM((PW,), jnp.int32),
            pltpu.VMEM((PW + VL,), jnp.int32),
            pltpu.VMEM((SROWS * D,), jnp.float32),
            pltpu.VMEM(((CHUNK + 4) * D,), jnp.float32),
            pltpu.VMEM(((CHUNK + 4) * D,), jnp.float32),
            pltpu.VMEM((VL,), jnp.int32),
            pltpu.SemaphoreType.DMA,
            pltpu.SemaphoreType.DMA,
        ],
    )


def kernel(x, duration, max_len):
    out_flat, mel2 = _regulate()(x.reshape(B * T * D),
                                 duration.astype(jnp.int32),
                                 jnp.zeros((PW,), jnp.int32))
    return out_flat.reshape(B, L, D), mel2[:, 0]


# speedup vs baseline: 1.7400x; 1.0131x over previous
import functools

import jax
import jax.numpy as jnp
from jax import lax
from jax.experimental import pallas as pl
from jax.experimental.pallas import tpu as pltpu
from jax.experimental.pallas import tpu_sc as plsc

B, T, D = 16, 1024, 256
L = 8192
NC, NS = 2, 16
NW = NC * NS
PW = B * L // NW
CHUNK = 128
NCHUNK = PW // CHUNK
SROWS = CHUNK + 8
VL = 16
DV = D // VL


def _body(x_hbm, dur_hbm, z_hbm, out_hbm, mel_hbm,
          dur_v, a_v, idx_v, sbuf, ob0, ob1, mel_v,
          cum_v, bp_v, cnt_v, bmp_v,
          gsem, wsem):
    cid = lax.axis_index("c")
    sid = lax.axis_index("s")
    wid = sid * NC + cid
    b = wid // 2
    half = wid % 2
    p0 = half * (L // 2)
    lane = jnp.arange(VL, dtype=jnp.int32)

    pltpu.sync_copy(dur_hbm.at[b], dur_v.at[pl.ds(0, T)])
    dur_v[pl.ds(T, VL)] = jnp.zeros((VL,), jnp.int32)

    pltpu.sync_copy(z_hbm, a_v)

    NB = T // VL
    NI = PW // VL
    one_lane = lane == 0

    @plsc.parallel_loop(0, NB, step=1, unroll=4)
    def _s1(j):
        v = dur_v[pl.ds(j * VL, VL)]
        s = plsc.cumsum(v)
        cum_v[pl.ds(j * VL, VL)] = s
        plsc.store_scatter(bp_v, (jnp.full((VL,), j, jnp.int32),),
                           jnp.full((VL,), 1, jnp.int32) * jnp.max(s),
                           mask=one_lane)

    @plsc.parallel_loop(0, NB // VL, step=1, carry=jnp.int32(0))
    def _s2(k, carry):
        v = bp_v[pl.ds(k * VL, VL)]
        s = plsc.cumsum(v) + carry
        bp_v[pl.ds(k * VL, VL)] = s
        return jnp.max(s)

    total = _s2
    bp_v[pl.ds(NB, VL)] = jnp.full((VL,), total, jnp.int32)

    def block_prefix(ref, j):
        return jnp.min(ref[pl.ds(j, VL)])

    @plsc.parallel_loop(0, NB, step=1, unroll=4)
    def _s3(j):
        pb = jnp.where(j == 0, 0, block_prefix(bp_v, jnp.maximum(j - 1, 0)))
        s = cum_v[pl.ds(j * VL, VL)] + pb
        i_vec = lane + j * VL
        d_next = dur_v[pl.ds(j * VL + 1, VL)]
        run_end = (d_next != 0) | (i_vec == T - 1)
        local = s - p0
        m = run_end & (local >= 0) & (local < PW)
        plsc.store_scatter(a_v, (jnp.where(m, local, 0),), i_vec + 1, mask=m)
        cnt = jnp.sum(jnp.where(s < p0, 1, 0).astype(jnp.int32))
        plsc.store_scatter(cnt_v, (jnp.full((VL,), j, jnp.int32),),
                           jnp.full((VL,), 1, jnp.int32) * cnt,
                           mask=one_lane)

    base = jnp.int32(0)
    for k in range(NB // VL):
        base = base + jnp.sum(cnt_v[pl.ds(k * VL, VL)])

    rowbase = b * T

    @plsc.parallel_loop(0, NI, step=1, unroll=4)
    def _w1(i):
        v = a_v[pl.ds(i * VL, VL)]
        s = plsc.cummax(v)
        idx_v[pl.ds(i * VL, VL)] = s
        plsc.store_scatter(bmp_v, (jnp.full((VL,), i, jnp.int32),),
                           jnp.full((VL,), 1, jnp.int32) * jnp.max(s),
                           mask=one_lane)

    @plsc.parallel_loop(0, NI // VL, step=1, carry=base)
    def _w2(k, carry):
        v = bmp_v[pl.ds(k * VL, VL)]
        s = jnp.maximum(plsc.cummax(v), carry)
        bmp_v[pl.ds(k * VL, VL)] = s
        return jnp.max(s)

    bmp_v[pl.ds(NI, VL)] = jnp.full((VL,), jnp.int32(T))

    @plsc.parallel_loop(0, NI, step=1, unroll=4)
    def _w3(i):
        pb = jnp.where(i == 0, base,
                       block_prefix(bmp_v, jnp.maximum(i - 1, 0)))
        s = jnp.maximum(idx_v[pl.ds(i * VL, VL)], pb)
        idx_v[pl.ds(i * VL, VL)] = jnp.minimum(s, T - 1) + rowbase

    idx_v[pl.ds(PW, VL)] = jnp.full((VL,), rowbase + T - 1, jnp.int32)

    @pl.when(half == 0)
    def _():
        mel_v[...] = jnp.full((VL,), total, jnp.int32)
        pltpu.sync_copy(mel_v, mel_hbm.at[b])

    n_valid = jnp.clip(total - p0, 0, PW)
    row0 = wid * PW

    def src_row(p):
        return jnp.min(idx_v[pl.ds(p, VL)])

    def drain_write():
        pltpu.make_async_copy(
            ob0.at[pl.ds(0, CHUNK * D)],
            out_hbm.at[pl.ds(row0 * D, CHUNK * D)], wsem).wait()

    def do_chunk(c, ob):
        c_lo = c * CHUNK
        r = jnp.clip(n_valid - c_lo, 0, CHUNK)
        lo_g = src_row(c_lo)
        hi_g = src_row(c_lo + jnp.maximum(r - 1, 0))
        span = hi_g - lo_g + 1
        start = pl.multiple_of(
            jnp.minimum((lo_g // 8) * 8, B * T - SROWS), 8)

        @pl.when((r > 0) & (span <= CHUNK))
        def _():
            pltpu.async_copy(
                x_hbm.at[pl.ds(start * D, SROWS * D)], sbuf, gsem).wait()

            @plsc.parallel_loop(0, r, step=1, unroll=4)
            def expand(p):
                so = jnp.clip(src_row(c_lo + p) - start, 0, SROWS - 1) * D
                po = p * D
                for d in range(DV):
                    ob[pl.ds(po + d * VL, VL)] = sbuf[pl.ds(so + d * VL, VL)]

        @pl.when((r > 0) & (span > CHUNK))
        def _():
            def row_copy(p, _):
                g = src_row(c_lo + p)
                pltpu.sync_copy(x_hbm.at[pl.ds(g * D, D)],
                                ob.at[pl.ds(p * D, D)])
                return 0
            lax.fori_loop(0, r, row_copy, 0)

        @plsc.parallel_loop(r, CHUNK, step=1, unroll=4)
        def zrow(p):
            po = p * D
            for d in range(DV):
                ob[pl.ds(po + d * VL, VL)] = jnp.zeros((VL,), jnp.float32)

        pltpu.async_copy(
            ob.at[pl.ds(0, CHUNK * D)],
            out_hbm.at[pl.ds((row0 + c_lo) * D, CHUNK * D)], wsem)

    def pair_body(cc, _):
        @pl.when(cc > 0)
        def _():
            drain_write()
            drain_write()
        do_chunk(cc * 2, ob0)
        do_chunk(cc * 2 + 1, ob1)
        return 0

    lax.fori_loop(0, NCHUNK // 2, pair_body, 0)
    drain_write()
    drain_write()


@functools.cache
def _regulate():
    return pl.kernel(
        _body,
        out_type=[
            jax.ShapeDtypeStruct((B * L * D,), jnp.float32),
            jax.ShapeDtypeStruct((B, VL), jnp.int32),
        ],
        name="length_regulator",
        mesh=plsc.VectorSubcoreMesh(core_axis_name="c", subcore_axis_name="s",
                                    num_cores=NC, num_subcores=NS),
        compiler_params=pltpu.CompilerParams(needs_layout_passes=False),
        scratch_types=[
            pltpu.VMEM((T + VL,), jnp.int32),
            pltpu.VMEM((PW,), jnp.int32),
            pltpu.VMEM((PW + VL,), jnp.int32),
            pltpu.VMEM((SROWS * D,), jnp.float32),
            pltpu.VMEM(((CHUNK + 4) * D,), jnp.float32),
            pltpu.VMEM(((CHUNK + 4) * D,), jnp.float32),
            pltpu.VMEM((VL,), jnp.int32),
            pltpu.VMEM((T,), jnp.int32),
            pltpu.VMEM((T // VL + VL,), jnp.int32),
            pltpu.VMEM((T // VL,), jnp.int32),
            pltpu.VMEM((PW // VL + VL,), jnp.int32),
            pltpu.SemaphoreType.DMA,
            pltpu.SemaphoreType.DMA,
        ],
    )


def kernel(x, duration, max_len):
    out_flat, mel2 = _regulate()(x.reshape(B * T * D),
                                 duration.astype(jnp.int32),
                                 jnp.zeros((PW,), jnp.int32))
    return out_flat.reshape(B, L, D), mel2[:, 0]


# speedup vs baseline: 3.0534x; 1.7548x over previous
import functools

import jax
import jax.numpy as jnp
from jax import lax
from jax.experimental import pallas as pl
from jax.experimental.pallas import tpu as pltpu
from jax.experimental.pallas import tpu_sc as plsc

B, T, D = 16, 1024, 256
L = 8192
NC, NS = 2, 16
NW = NC * NS
PW = B * L // NW
CHUNK = 128
NCHUNK = PW // CHUNK
SROWS = CHUNK + 8
VL = 16
DV = D // VL


def _body(x_hbm, dur_hbm, z_hbm, out_hbm, mel_hbm,
          dur_v, a_v, idx_v, sbuf, ob0, ob1, mel_v,
          cum_v, bp_v, cnt_v, bmp_v,
          gsem, wsem):
    cid = lax.axis_index("c")
    sid = lax.axis_index("s")
    wid = sid * NC + cid
    b = wid // 2
    half = wid % 2
    p0 = half * (L // 2)
    lane = jnp.arange(VL, dtype=jnp.int32)

    pltpu.sync_copy(dur_hbm.at[b], dur_v.at[pl.ds(0, T)])
    dur_v[pl.ds(T, VL)] = jnp.zeros((VL,), jnp.int32)

    pltpu.sync_copy(z_hbm, a_v)

    if True:
        @pl.when(half == 0)
        def _():
            mel_v[...] = jnp.full((VL,), 0, jnp.int32)
            pltpu.sync_copy(mel_v, mel_hbm.at[b])
        return
    NB = T // VL
    NI = PW // VL
    one_lane = lane == 0

    @plsc.parallel_loop(0, NB, step=1, unroll=4)
    def _s1(j):
        v = dur_v[pl.ds(j * VL, VL)]
        s = plsc.cumsum(v)
        cum_v[pl.ds(j * VL, VL)] = s
        plsc.store_scatter(bp_v, (jnp.full((VL,), j, jnp.int32),),
                           jnp.full((VL,), 1, jnp.int32) * jnp.max(s),
                           mask=one_lane)

    @plsc.parallel_loop(0, NB // VL, step=1, carry=jnp.int32(0))
    def _s2(k, carry):
        v = bp_v[pl.ds(k * VL, VL)]
        s = plsc.cumsum(v) + carry
        bp_v[pl.ds(k * VL, VL)] = s
        return jnp.max(s)

    total = _s2
    bp_v[pl.ds(NB, VL)] = jnp.full((VL,), total, jnp.int32)

    def block_prefix(ref, j):
        return jnp.min(ref[pl.ds(j, VL)])

    @plsc.parallel_loop(0, NB, step=1, unroll=4)
    def _s3(j):
        pb = jnp.where(j == 0, 0, block_prefix(bp_v, jnp.maximum(j - 1, 0)))
        s = cum_v[pl.ds(j * VL, VL)] + pb
        i_vec = lane + j * VL
        d_next = dur_v[pl.ds(j * VL + 1, VL)]
        run_end = (d_next != 0) | (i_vec == T - 1)
        local = s - p0
        m = run_end & (local >= 0) & (local < PW)
        plsc.store_scatter(a_v, (jnp.where(m, local, 0),), i_vec + 1, mask=m)
        cnt = jnp.sum(jnp.where(s < p0, 1, 0).astype(jnp.int32))
        plsc.store_scatter(cnt_v, (jnp.full((VL,), j, jnp.int32),),
                           jnp.full((VL,), 1, jnp.int32) * cnt,
                           mask=one_lane)

    base = jnp.int32(0)
    for k in range(NB // VL):
        base = base + jnp.sum(cnt_v[pl.ds(k * VL, VL)])

    rowbase = b * T

    @plsc.parallel_loop(0, NI, step=1, unroll=4)
    def _w1(i):
        v = a_v[pl.ds(i * VL, VL)]
        s = plsc.cummax(v)
        idx_v[pl.ds(i * VL, VL)] = s
        plsc.store_scatter(bmp_v, (jnp.full((VL,), i, jnp.int32),),
                           jnp.full((VL,), 1, jnp.int32) * jnp.max(s),
                           mask=one_lane)

    @plsc.parallel_loop(0, NI // VL, step=1, carry=base)
    def _w2(k, carry):
        v = bmp_v[pl.ds(k * VL, VL)]
        s = jnp.maximum(plsc.cummax(v), carry)
        bmp_v[pl.ds(k * VL, VL)] = s
        return jnp.max(s)

    bmp_v[pl.ds(NI, VL)] = jnp.full((VL,), jnp.int32(T))

    @plsc.parallel_loop(0, NI, step=1, unroll=4)
    def _w3(i):
        pb = jnp.where(i == 0, base,
                       block_prefix(bmp_v, jnp.maximum(i - 1, 0)))
        s = jnp.maximum(idx_v[pl.ds(i * VL, VL)], pb)
        idx_v[pl.ds(i * VL, VL)] = jnp.minimum(s, T - 1) + rowbase

    idx_v[pl.ds(PW, VL)] = jnp.full((VL,), rowbase + T - 1, jnp.int32)

    @pl.when(half == 0)
    def _():
        mel_v[...] = jnp.full((VL,), total, jnp.int32)
        pltpu.sync_copy(mel_v, mel_hbm.at[b])

    n_valid = jnp.clip(total - p0, 0, PW)
    row0 = wid * PW

    def src_row(p):
        return jnp.min(idx_v[pl.ds(p, VL)])

    def drain_write():
        pltpu.make_async_copy(
            ob0.at[pl.ds(0, CHUNK * D)],
            out_hbm.at[pl.ds(row0 * D, CHUNK * D)], wsem).wait()

    def do_chunk(c, ob):
        c_lo = c * CHUNK
        r = jnp.clip(n_valid - c_lo, 0, CHUNK)
        lo_g = src_row(c_lo)
        hi_g = src_row(c_lo + jnp.maximum(r - 1, 0))
        span = hi_g - lo_g + 1
        start = pl.multiple_of(
            jnp.minimum((lo_g // 8) * 8, B * T - SROWS), 8)

        @pl.when((r > 0) & (span <= CHUNK))
        def _():
            pltpu.async_copy(
                x_hbm.at[pl.ds(start * D, SROWS * D)], sbuf, gsem).wait()

            @plsc.parallel_loop(0, r, step=1, unroll=4)
            def expand(p):
                so = jnp.clip(src_row(c_lo + p) - start, 0, SROWS - 1) * D
                po = p * D
                for d in range(DV):
                    ob[pl.ds(po + d * VL, VL)] = sbuf[pl.ds(so + d * VL, VL)]

        @pl.when((r > 0) & (span > CHUNK))
        def _():
            def row_copy(p, _):
                g = src_row(c_lo + p)
                pltpu.sync_copy(x_hbm.at[pl.ds(g * D, D)],
                                ob.at[pl.ds(p * D, D)])
                return 0
            lax.fori_loop(0, r, row_copy, 0)

        @plsc.parallel_loop(r, CHUNK, step=1, unroll=4)
        def zrow(p):
            po = p * D
            for d in range(DV):
                ob[pl.ds(po + d * VL, VL)] = jnp.zeros((VL,), jnp.float32)

        pltpu.async_copy(
            ob.at[pl.ds(0, CHUNK * D)],
            out_hbm.at[pl.ds((row0 + c_lo) * D, CHUNK * D)], wsem)

    def pair_body(cc, _):
        @pl.when(cc > 0)
        def _():
            drain_write()
            drain_write()
        do_chunk(cc * 2, ob0)
        do_chunk(cc * 2 + 1, ob1)
        return 0

    lax.fori_loop(0, NCHUNK // 2, pair_body, 0)
    drain_write()
    drain_write()


@functools.cache
def _regulate():
    return pl.kernel(
        _body,
        out_type=[
            jax.ShapeDtypeStruct((B * L * D,), jnp.float32),
            jax.ShapeDtypeStruct((B, VL), jnp.int32),
        ],
        name="length_regulator",
        mesh=plsc.VectorSubcoreMesh(core_axis_name="c", subcore_axis_name="s",
                                    num_cores=NC, num_subcores=NS),
        compiler_params=pltpu.CompilerParams(needs_layout_passes=False),
        scratch_types=[
            pltpu.VMEM((T + VL,), jnp.int32),
            pltpu.VMEM((PW,), jnp.int32),
            pltpu.VMEM((PW + VL,), jnp.int32),
            pltpu.VMEM((SROWS * D,), jnp.float32),
            pltpu.VMEM(((CHUNK + 4) * D,), jnp.float32),
            pltpu.VMEM(((CHUNK + 4) * D,), jnp.float32),
            pltpu.VMEM((VL,), jnp.int32),
            pltpu.VMEM((T,), jnp.int32),
            pltpu.VMEM((T // VL + VL,), jnp.int32),
            pltpu.VMEM((T // VL,), jnp.int32),
            pltpu.VMEM((PW // VL + VL,), jnp.int32),
            pltpu.SemaphoreType.DMA,
            pltpu.SemaphoreType.DMA,
        ],
    )


def kernel(x, duration, max_len):
    out_flat, mel2 = _regulate()(x.reshape(B * T * D),
                                 duration.astype(jnp.int32),
                                 jnp.zeros((PW,), jnp.int32))
    return out_flat.reshape(B, L, D), mel2[:, 0]


# speedup vs baseline: 3.1585x; 1.0344x over previous
import functools

import jax
import jax.numpy as jnp
from jax import lax
from jax.experimental import pallas as pl
from jax.experimental.pallas import tpu as pltpu
from jax.experimental.pallas import tpu_sc as plsc

B, T, D = 16, 1024, 256
L = 8192
NC, NS = 2, 16
NW = NC * NS
PW = B * L // NW
CHUNK = 128
NCHUNK = PW // CHUNK
SROWS = CHUNK + 8
NWIN = (T + CHUNK - 1) // CHUNK
VL = 16
DV = D // VL


def _body(x_hbm, dur_hbm, z_hbm, out_hbm, mel_hbm,
          dur_v, a_v, idx_v, sbuf, ob0, ob1, mel_v,
          cum_v, bp_v, cnt_v, bmp_v,
          gsem, wsem):
    cid = lax.axis_index("c")
    sid = lax.axis_index("s")
    wid = sid * NC + cid
    b = wid // 2
    half = wid % 2
    p0 = half * (L // 2)
    lane = jnp.arange(VL, dtype=jnp.int32)

    pltpu.sync_copy(dur_hbm.at[b], dur_v.at[pl.ds(0, T)])
    dur_v[pl.ds(T, VL)] = jnp.zeros((VL,), jnp.int32)

    pltpu.sync_copy(z_hbm, a_v)

    NB = T // VL
    NI = PW // VL
    one_lane = lane == 0

    @plsc.parallel_loop(0, NB, step=1, unroll=4)
    def _s1(j):
        v = dur_v[pl.ds(j * VL, VL)]
        s = plsc.cumsum(v)
        cum_v[pl.ds(j * VL, VL)] = s
        plsc.store_scatter(bp_v, (jnp.full((VL,), j, jnp.int32),),
                           jnp.full((VL,), 1, jnp.int32) * jnp.max(s),
                           mask=one_lane)

    @plsc.parallel_loop(0, NB // VL, step=1, carry=jnp.int32(0))
    def _s2(k, carry):
        v = bp_v[pl.ds(k * VL, VL)]
        s = plsc.cumsum(v) + carry
        bp_v[pl.ds(k * VL, VL)] = s
        return jnp.max(s)

    total = _s2
    bp_v[pl.ds(NB, VL)] = jnp.full((VL,), total, jnp.int32)

    def block_prefix(ref, j):
        return jnp.min(ref[pl.ds(j, VL)])

    @plsc.parallel_loop(0, NB, step=1, unroll=4)
    def _s3(j):
        pb = jnp.where(j == 0, 0, block_prefix(bp_v, jnp.maximum(j - 1, 0)))
        s = cum_v[pl.ds(j * VL, VL)] + pb
        i_vec = lane + j * VL
        d_next = dur_v[pl.ds(j * VL + 1, VL)]
        run_end = (d_next != 0) | (i_vec == T - 1)
        local = s - p0
        m = run_end & (local >= 0) & (local < PW)
        plsc.store_scatter(a_v, (jnp.where(m, local, 0),), i_vec + 1, mask=m)
        cnt = jnp.sum(jnp.where(s < p0, 1, 0).astype(jnp.int32))
        plsc.store_scatter(cnt_v, (jnp.full((VL,), j, jnp.int32),),
                           jnp.full((VL,), 1, jnp.int32) * cnt,
                           mask=one_lane)

    base = jnp.int32(0)
    for k in range(NB // VL):
        base = base + jnp.sum(cnt_v[pl.ds(k * VL, VL)])

    rowbase = b * T

    @plsc.parallel_loop(0, NI, step=1, unroll=4)
    def _w1(i):
        v = a_v[pl.ds(i * VL, VL)]
        s = plsc.cummax(v)
        idx_v[pl.ds(i * VL, VL)] = s
        plsc.store_scatter(bmp_v, (jnp.full((VL,), i, jnp.int32),),
                           jnp.full((VL,), 1, jnp.int32) * jnp.max(s),
                           mask=one_lane)

    @plsc.parallel_loop(0, NI // VL, step=1, carry=base)
    def _w2(k, carry):
        v = bmp_v[pl.ds(k * VL, VL)]
        s = jnp.maximum(plsc.cummax(v), carry)
        bmp_v[pl.ds(k * VL, VL)] = s
        return jnp.max(s)

    bmp_v[pl.ds(NI, VL)] = jnp.full((VL,), jnp.int32(T))

    @plsc.parallel_loop(0, NI, step=1, unroll=4)
    def _w3(i):
        pb = jnp.where(i == 0, base,
                       block_prefix(bmp_v, jnp.maximum(i - 1, 0)))
        s = jnp.maximum(idx_v[pl.ds(i * VL, VL)], pb)
        idx_v[pl.ds(i * VL, VL)] = jnp.minimum(s, T - 1) + rowbase

    idx_v[pl.ds(PW, VL)] = jnp.full((VL,), rowbase + T - 1, jnp.int32)

    @pl.when(half == 0)
    def _():
        mel_v[...] = jnp.full((VL,), total, jnp.int32)
        pltpu.sync_copy(mel_v, mel_hbm.at[b])

    n_valid = jnp.clip(total - p0, 0, PW)
    row0 = wid * PW

    def src_row(p):
        return jnp.min(idx_v[pl.ds(p, VL)])

    def drain_write():
        pltpu.make_async_copy(
            ob0, out_hbm.at[pl.ds(row0, CHUNK)], wsem).wait()

    def do_chunk(c, ob):
        c_lo = c * CHUNK
        r = jnp.clip(n_valid - c_lo, 0, CHUNK)
        lo_g = src_row(c_lo)
        hi_g = src_row(c_lo + jnp.maximum(r - 1, 0))
        start = pl.multiple_of(
            jnp.minimum((lo_g // 8) * 8, B * T - SROWS), 8)
        fits = hi_g - start < SROWS

        @pl.when(r > 0)
        def _():
            pltpu.async_copy(
                x_hbm.at[pl.ds(start, SROWS)], sbuf, gsem).wait()

            @plsc.parallel_loop(0, r, step=1, unroll=4)
            def expand(p):
                so = jnp.clip(src_row(c_lo + p) - start, 0, SROWS - 1)
                for d in range(DV):
                    ob[p, pl.ds(d * VL, VL)] = sbuf[so, pl.ds(d * VL, VL)]

        @pl.when((r > 0) & jnp.logical_not(fits))
        def _():
            def repair(w, _):
                wstart = pl.multiple_of(
                    jnp.minimum(start + w * CHUNK, B * T - SROWS), 8)

                @pl.when(wstart <= hi_g)
                def _():
                    pltpu.async_copy(
                        x_hbm.at[pl.ds(wstart, SROWS)], sbuf, gsem).wait()

                    def fix(p, _):
                        so = src_row(c_lo + p) - wstart

                        @pl.when((so >= 0) & (so < SROWS))
                        def _():
                            for d in range(DV):
                                ob[p, pl.ds(d * VL, VL)] = \
                                    sbuf[so, pl.ds(d * VL, VL)]
                        return 0
                    lax.fori_loop(0, r, fix, 0)
                return 0
            lax.fori_loop(1, NWIN, repair, 0)

        @plsc.parallel_loop(r, CHUNK, step=1, unroll=4)
        def zrow(p):
            for d in range(DV):
                ob[p, pl.ds(d * VL, VL)] = jnp.zeros((VL,), jnp.float32)

        pltpu.async_copy(
            ob, out_hbm.at[pl.ds(row0 + c_lo, CHUNK)], wsem)

    def pair_body(cc, _):
        @pl.when(cc > 0)
        def _():
            drain_write()
            drain_write()
        do_chunk(cc * 2, ob0)
        do_chunk(cc * 2 + 1, ob1)
        return 0

    lax.fori_loop(0, NCHUNK // 2, pair_body, 0)
    drain_write()
    drain_write()


@functools.cache
def _regulate():
    return pl.kernel(
        _body,
        out_type=[
            jax.ShapeDtypeStruct((B * L, D), jnp.float32),
            jax.ShapeDtypeStruct((B, VL), jnp.int32),
        ],
        name="length_regulator",
        mesh=plsc.VectorSubcoreMesh(core_axis_name="c", subcore_axis_name="s",
                                    num_cores=NC, num_subcores=NS),
        compiler_params=pltpu.CompilerParams(needs_layout_passes=False),
        scratch_types=[
            pltpu.VMEM((T + VL,), jnp.int32),
            pltpu.VMEM((PW,), jnp.int32),
            pltpu.VMEM((PW + VL,), jnp.int32),
            pltpu.VMEM((SROWS, D), jnp.float32),
            pltpu.VMEM((CHUNK, D), jnp.float32),
            pltpu.VMEM((CHUNK, D), jnp.float32),
            pltpu.VMEM((VL,), jnp.int32),
            pltpu.VMEM((T,), jnp.int32),
            pltpu.VMEM((T // VL + VL,), jnp.int32),
            pltpu.VMEM((T // VL,), jnp.int32),
            pltpu.VMEM((PW // VL + VL,), jnp.int32),
            pltpu.SemaphoreType.DMA,
            pltpu.SemaphoreType.DMA,
        ],
    )


def kernel(x, duration, max_len):
    out_flat, mel2 = _regulate()(x.reshape(B * T, D),
                                 duration.astype(jnp.int32),
                                 jnp.zeros((PW,), jnp.int32))
    return out_flat.reshape(B, L, D), mel2[:, 0]


# speedup vs baseline: 4.0305x; 1.2761x over previous
import functools

import jax
import jax.numpy as jnp
from jax import lax
from jax.experimental import pallas as pl
from jax.experimental.pallas import tpu as pltpu
from jax.experimental.pallas import tpu_sc as plsc

B, T, D = 16, 1024, 256
L = 8192
NC, NS = 2, 16
NW = NC * NS
PW = B * L // NW
CHUNK = 128
NCHUNK = PW // CHUNK
SROWS = CHUNK + 8
NWIN = (T + CHUNK - 1) // CHUNK
VL = 16
DV = D // VL


def _body(x_hbm, dur_hbm, z_hbm, out_hbm, mel_hbm,
          xch, dur_v, a_v, idx_v, sbuf, ob0, ob1, mel_v,
          cum_v, bp_v, cnt_v, bmp_v,
          gsem, wsem):
    cid = lax.axis_index("c")
    sid = lax.axis_index("s")
    b = cid * (B // NC) + sid // 2
    half = sid % 2
    p0 = half * (L // 2)
    lane = jnp.arange(VL, dtype=jnp.int32)

    pltpu.sync_copy(dur_hbm.at[b], dur_v.at[pl.ds(0, T)])
    dur_v[pl.ds(T, VL)] = jnp.zeros((VL,), jnp.int32)

    pltpu.sync_copy(z_hbm, a_v)

    NB = T // VL
    NI = PW // VL
    one_lane = lane == 0

    @plsc.parallel_loop(0, NB, step=1, unroll=4)
    def _s1(j):
        v = dur_v[pl.ds(j * VL, VL)]
        s = plsc.cumsum(v)
        cum_v[pl.ds(j * VL, VL)] = s
        plsc.store_scatter(bp_v, (jnp.full((VL,), j, jnp.int32),),
                           jnp.full((VL,), 1, jnp.int32) * jnp.max(s),
                           mask=one_lane)

    @plsc.parallel_loop(0, NB // VL, step=1, carry=jnp.int32(0))
    def _s2(k, carry):
        v = bp_v[pl.ds(k * VL, VL)]
        s = plsc.cumsum(v) + carry
        bp_v[pl.ds(k * VL, VL)] = s
        return jnp.max(s)

    total = _s2
    bp_v[pl.ds(NB, VL)] = jnp.full((VL,), total, jnp.int32)

    def block_prefix(ref, j):
        return jnp.min(ref[pl.ds(j, VL)])

    @plsc.parallel_loop(0, NB, step=1, unroll=4)
    def _s3(j):
        pb = jnp.where(j == 0, 0, block_prefix(bp_v, jnp.maximum(j - 1, 0)))
        s = cum_v[pl.ds(j * VL, VL)] + pb
        i_vec = lane + j * VL
        d_next = dur_v[pl.ds(j * VL + 1, VL)]
        run_end = (d_next != 0) | (i_vec == T - 1)
        local = s - p0
        m = run_end & (local >= 0) & (local < PW)
        plsc.store_scatter(a_v, (jnp.where(m, local, 0),), i_vec + 1, mask=m)
        cnt = jnp.sum(jnp.where(s < p0, 1, 0).astype(jnp.int32))
        plsc.store_scatter(cnt_v, (jnp.full((VL,), j, jnp.int32),),
                           jnp.full((VL,), 1, jnp.int32) * cnt,
                           mask=one_lane)

    base = jnp.int32(0)
    for k in range(NB // VL):
        base = base + jnp.sum(cnt_v[pl.ds(k * VL, VL)])

    rowbase = b * T

    w_off = half * PW

    @plsc.parallel_loop(0, NI, step=1, unroll=4)
    def _w1(i):
        v = a_v[pl.ds(i * VL, VL)]
        s = plsc.cummax(v)
        idx_v[pl.ds(w_off + i * VL, VL)] = s
        plsc.store_scatter(bmp_v, (jnp.full((VL,), i, jnp.int32),),
                           jnp.full((VL,), 1, jnp.int32) * jnp.max(s),
                           mask=one_lane)

    @plsc.parallel_loop(0, NI // VL, step=1, carry=base)
    def _w2(k, carry):
        v = bmp_v[pl.ds(k * VL, VL)]
        s = jnp.maximum(plsc.cummax(v), carry)
        bmp_v[pl.ds(k * VL, VL)] = s
        return jnp.max(s)

    bmp_v[pl.ds(NI, VL)] = jnp.full((VL,), jnp.int32(T))

    @plsc.parallel_loop(0, NI, step=1, unroll=4)
    def _w3(i):
        pb = jnp.where(i == 0, base,
                       block_prefix(bmp_v, jnp.maximum(i - 1, 0)))
        s = jnp.maximum(idx_v[pl.ds(w_off + i * VL, VL)], pb)
        idx_v[pl.ds(w_off + i * VL, VL)] = jnp.minimum(s, T - 1) + rowbase

    pltpu.sync_copy(idx_v.at[pl.ds(w_off, PW)], xch.at[sid])
    plsc.subcore_barrier()
    pltpu.sync_copy(xch.at[sid + 1 - 2 * half],
                    idx_v.at[pl.ds(PW - w_off, PW)])

    idx_v[pl.ds(2 * PW, VL)] = jnp.full((VL,), rowbase + T - 1, jnp.int32)

    @pl.when(half == 0)
    def _():
        mel_v[...] = jnp.full((VL,), total, jnp.int32)
        pltpu.sync_copy(mel_v, mel_hbm.at[b])

    brow = b * L

    def src_row(p):
        return jnp.min(idx_v[pl.ds(p, VL)])

    def drain_write():
        pltpu.make_async_copy(
            ob0, out_hbm.at[pl.ds(brow, CHUNK)], wsem).wait()

    def do_chunk(cg, ob):
        c_lo = cg * CHUNK
        r = jnp.clip(total - c_lo, 0, CHUNK)
        lo_g = src_row(c_lo)
        hi_g = src_row(c_lo + jnp.maximum(r - 1, 0))
        start = pl.multiple_of(
            jnp.minimum((lo_g // 8) * 8, B * T - SROWS), 8)
        fits = hi_g - start < SROWS

        @pl.when(r > 0)
        def _():
            pltpu.async_copy(
                x_hbm.at[pl.ds(start, SROWS)], sbuf, gsem).wait()

            @plsc.parallel_loop(0, r, step=1, unroll=4)
            def expand(p):
                so = jnp.clip(src_row(c_lo + p) - start, 0, SROWS - 1)
                for d in range(DV):
                    ob[p, pl.ds(d * VL, VL)] = sbuf[so, pl.ds(d * VL, VL)]

        @pl.when((r > 0) & jnp.logical_not(fits))
        def _():
            def repair(w, _):
                wstart = pl.multiple_of(
                    jnp.minimum(start + w * CHUNK, B * T - SROWS), 8)

                @pl.when(wstart <= hi_g)
                def _():
                    pltpu.async_copy(
                        x_hbm.at[pl.ds(wstart, SROWS)], sbuf, gsem).wait()

                    def fix(p, _):
                        so = src_row(c_lo + p) - wstart

                        @pl.when((so >= 0) & (so < SROWS))
                        def _():
                            for d in range(DV):
                                ob[p, pl.ds(d * VL, VL)] = \
                                    sbuf[so, pl.ds(d * VL, VL)]
                        return 0
                    lax.fori_loop(0, r, fix, 0)
                return 0
            lax.fori_loop(1, NWIN, repair, 0)

        @plsc.parallel_loop(r, CHUNK, step=1, unroll=4)
        def zrow(p):
            for d in range(DV):
                ob[p, pl.ds(d * VL, VL)] = jnp.zeros((VL,), jnp.float32)

        pltpu.async_copy(
            ob, out_hbm.at[pl.ds(brow + c_lo, CHUNK)], wsem)

    def pair_body(cc, _):
        @pl.when(cc > 0)
        def _():
            drain_write()
            drain_write()
        do_chunk(cc * 4 + half, ob0)
        do_chunk(cc * 4 + 2 + half, ob1)
        return 0

    lax.fori_loop(0, NCHUNK // 2, pair_body, 0)
    drain_write()
    drain_write()


@functools.cache
def _regulate():
    return pl.kernel(
        _body,
        out_type=[
            jax.ShapeDtypeStruct((B * L, D), jnp.float32),
            jax.ShapeDtypeStruct((B, VL), jnp.int32),
        ],
        name="length_regulator",
        mesh=plsc.VectorSubcoreMesh(core_axis_name="c", subcore_axis_name="s",
                                    num_cores=NC, num_subcores=NS),
        compiler_params=pltpu.CompilerParams(needs_layout_passes=False),
        scratch_types=[
            pltpu.VMEM_SHARED((NS, PW), jnp.int32),
            pltpu.VMEM((T + VL,), jnp.int32),
            pltpu.VMEM((PW,), jnp.int32),
            pltpu.VMEM((2 * PW + VL,), jnp.int32),
            pltpu.VMEM((SROWS, D), jnp.float32),
            pltpu.VMEM((CHUNK, D), jnp.float32),
            pltpu.VMEM((CHUNK, D), jnp.float32),
            pltpu.VMEM((VL,), jnp.int32),
            pltpu.VMEM((T,), jnp.int32),
            pltpu.VMEM((T // VL + VL,), jnp.int32),
            pltpu.VMEM((T // VL,), jnp.int32),
            pltpu.VMEM((PW // VL + VL,), jnp.int32),
            pltpu.SemaphoreType.DMA,
            pltpu.SemaphoreType.DMA,
        ],
    )


def kernel(x, duration, max_len):
    out_flat, mel2 = _regulate()(x.reshape(B * T, D),
                                 duration.astype(jnp.int32),
                                 jnp.zeros((PW,), jnp.int32))
    return out_flat.reshape(B, L, D), mel2[:, 0]


# speedup vs baseline: 4.5248x; 1.1226x over previous
import functools

import jax
import jax.numpy as jnp
from jax import lax
from jax.experimental import pallas as pl
from jax.experimental.pallas import tpu as pltpu
from jax.experimental.pallas import tpu_sc as plsc

B, T, D = 16, 1024, 256
L = 8192
NC, NS = 2, 16
NW = NC * NS
PW = B * L // NW
CHUNK = 128
NCHUNK = PW // CHUNK
SROWS = CHUNK + 8
SSMALL = 48
NWIN = (T + CHUNK - 1) // CHUNK
VL = 16
DV = D // VL


def _body(x_hbm, dur_hbm, z_hbm, out_hbm, mel_hbm,
          xch, dur_v, a_v, idx_v, sbuf, ob0, ob1, mel_v,
          cum_v, bp_v, cnt_v, bmp_v,
          gsem, wsem):
    cid = lax.axis_index("c")
    sid = lax.axis_index("s")
    b = cid * (B // NC) + sid // 2
    half = sid % 2
    p0 = half * (L // 2)
    lane = jnp.arange(VL, dtype=jnp.int32)

    pltpu.sync_copy(dur_hbm.at[b], dur_v.at[pl.ds(0, T)])
    dur_v[pl.ds(T, VL)] = jnp.zeros((VL,), jnp.int32)

    pltpu.sync_copy(z_hbm, a_v)

    NB = T // VL
    NI = PW // VL
    one_lane = lane == 0

    @plsc.parallel_loop(0, NB, step=1, unroll=4)
    def _s1(j):
        v = dur_v[pl.ds(j * VL, VL)]
        s = plsc.cumsum(v)
        cum_v[pl.ds(j * VL, VL)] = s
        plsc.store_scatter(bp_v, (jnp.full((VL,), j, jnp.int32),),
                           jnp.full((VL,), 1, jnp.int32) * jnp.max(s),
                           mask=one_lane)

    @plsc.parallel_loop(0, NB // VL, step=1, carry=jnp.int32(0))
    def _s2(k, carry):
        v = bp_v[pl.ds(k * VL, VL)]
        s = plsc.cumsum(v) + carry
        bp_v[pl.ds(k * VL, VL)] = s
        return jnp.max(s)

    total = _s2
    bp_v[pl.ds(NB, VL)] = jnp.full((VL,), total, jnp.int32)

    def block_prefix(ref, j):
        return jnp.min(ref[pl.ds(j, VL)])

    @plsc.parallel_loop(0, NB, step=1, unroll=4)
    def _s3(j):
        pb = jnp.where(j == 0, 0, block_prefix(bp_v, jnp.maximum(j - 1, 0)))
        s = cum_v[pl.ds(j * VL, VL)] + pb
        i_vec = lane + j * VL
        d_next = dur_v[pl.ds(j * VL + 1, VL)]
        run_end = (d_next != 0) | (i_vec == T - 1)
        local = s - p0
        m = run_end & (local >= 0) & (local < PW)
        plsc.store_scatter(a_v, (jnp.where(m, local, 0),), i_vec + 1, mask=m)
        cnt = jnp.sum(jnp.where(s < p0, 1, 0).astype(jnp.int32))
        plsc.store_scatter(cnt_v, (jnp.full((VL,), j, jnp.int32),),
                           jnp.full((VL,), 1, jnp.int32) * cnt,
                           mask=one_lane)

    base = jnp.int32(0)
    for k in range(NB // VL):
        base = base + jnp.sum(cnt_v[pl.ds(k * VL, VL)])

    rowbase = b * T

    w_off = half * PW

    @plsc.parallel_loop(0, NI, step=1, unroll=4)
    def _w1(i):
        v = a_v[pl.ds(i * VL, VL)]
        s = plsc.cummax(v)
        idx_v[pl.ds(w_off + i * VL, VL)] = s
        plsc.store_scatter(bmp_v, (jnp.full((VL,), i, jnp.int32),),
                           jnp.full((VL,), 1, jnp.int32) * jnp.max(s),
                           mask=one_lane)

    @plsc.parallel_loop(0, NI // VL, step=1, carry=base)
    def _w2(k, carry):
        v = bmp_v[pl.ds(k * VL, VL)]
        s = jnp.maximum(plsc.cummax(v), carry)
        bmp_v[pl.ds(k * VL, VL)] = s
        return jnp.max(s)

    bmp_v[pl.ds(NI, VL)] = jnp.full((VL,), jnp.int32(T))

    @plsc.parallel_loop(0, NI, step=1, unroll=4)
    def _w3(i):
        pb = jnp.where(i == 0, base,
                       block_prefix(bmp_v, jnp.maximum(i - 1, 0)))
        s = jnp.maximum(idx_v[pl.ds(w_off + i * VL, VL)], pb)
        idx_v[pl.ds(w_off + i * VL, VL)] = jnp.minimum(s, T - 1) + rowbase

    pltpu.sync_copy(idx_v.at[pl.ds(w_off, PW)], xch.at[sid])
    plsc.subcore_barrier()
    pltpu.sync_copy(xch.at[sid + 1 - 2 * half],
                    idx_v.at[pl.ds(PW - w_off, PW)])

    idx_v[pl.ds(2 * PW, VL)] = jnp.full((VL,), rowbase + T - 1, jnp.int32)

    @pl.when(half == 0)
    def _():
        mel_v[...] = jnp.full((VL,), total, jnp.int32)
        pltpu.sync_copy(mel_v, mel_hbm.at[b])

    brow = b * L

    def src_row(p):
        return jnp.min(idx_v[pl.ds(p, VL)])

    def drain_write():
        pltpu.make_async_copy(
            ob0, out_hbm.at[pl.ds(brow, CHUNK)], wsem).wait()

    def do_chunk(cg, ob):
        c_lo = cg * CHUNK
        r = jnp.clip(total - c_lo, 0, CHUNK)
        lo_g = src_row(c_lo)
        hi_g = src_row(c_lo + jnp.maximum(r - 1, 0))
        start = pl.multiple_of(
            jnp.minimum((lo_g // 8) * 8, B * T - SROWS), 8)
        fits = hi_g - start < SROWS

        small = hi_g - start < SSMALL

        @pl.when((r > 0) & small)
        def _():
            pltpu.async_copy(
                x_hbm.at[pl.ds(start, SSMALL)],
                sbuf.at[pl.ds(0, SSMALL)], gsem).wait()

        @pl.when((r > 0) & jnp.logical_not(small))
        def _():
            pltpu.async_copy(
                x_hbm.at[pl.ds(start, SROWS)], sbuf, gsem).wait()

        @pl.when(r > 0)
        def _():

            @plsc.parallel_loop(0, r, step=1, unroll=4)
            def expand(p):
                so = jnp.clip(src_row(c_lo + p) - start, 0, SROWS - 1)
                for d in range(DV):
                    ob[p, pl.ds(d * VL, VL)] = sbuf[so, pl.ds(d * VL, VL)]

        @pl.when((r > 0) & jnp.logical_not(fits))
        def _():
            def repair(w, _):
                wstart = pl.multiple_of(
                    jnp.minimum(start + w * CHUNK, B * T - SROWS), 8)

                @pl.when(wstart <= hi_g)
                def _():
                    pltpu.async_copy(
                        x_hbm.at[pl.ds(wstart, SROWS)], sbuf, gsem).wait()

                    def fix(p, _):
                        so = src_row(c_lo + p) - wstart

                        @pl.when((so >= 0) & (so < SROWS))
                        def _():
                            for d in range(DV):
                                ob[p, pl.ds(d * VL, VL)] = \
                                    sbuf[so, pl.ds(d * VL, VL)]
                        return 0
                    lax.fori_loop(0, r, fix, 0)
                return 0
            lax.fori_loop(1, NWIN, repair, 0)

        @plsc.parallel_loop(r, CHUNK, step=1, unroll=4)
        def zrow(p):
            for d in range(DV):
                ob[p, pl.ds(d * VL, VL)] = jnp.zeros((VL,), jnp.float32)

        pltpu.async_copy(
            ob, out_hbm.at[pl.ds(brow + c_lo, CHUNK)], wsem)

    def pair_body(cc, _):
        @pl.when(cc > 0)
        def _():
            drain_write()
            drain_write()
        do_chunk(cc * 4 + half, ob0)
        do_chunk(cc * 4 + 2 + half, ob1)
        return 0

    lax.fori_loop(0, NCHUNK // 2, pair_body, 0)
    drain_write()
    drain_write()


@functools.cache
def _regulate():
    return pl.kernel(
        _body,
        out_type=[
            jax.ShapeDtypeStruct((B * L, D), jnp.float32),
            jax.ShapeDtypeStruct((B, VL), jnp.int32),
        ],
        name="length_regulator",
        mesh=plsc.VectorSubcoreMesh(core_axis_name="c", subcore_axis_name="s",
                                    num_cores=NC, num_subcores=NS),
        compiler_params=pltpu.CompilerParams(needs_layout_passes=False),
        scratch_types=[
            pltpu.VMEM_SHARED((NS, PW), jnp.int32),
            pltpu.VMEM((T + VL,), jnp.int32),
            pltpu.VMEM((PW,), jnp.int32),
            pltpu.VMEM((2 * PW + VL,), jnp.int32),
            pltpu.VMEM((SROWS, D), jnp.float32),
            pltpu.VMEM((CHUNK, D), jnp.float32),
            pltpu.VMEM((CHUNK, D), jnp.float32),
            pltpu.VMEM((VL,), jnp.int32),
            pltpu.VMEM((T,), jnp.int32),
            pltpu.VMEM((T // VL + VL,), jnp.int32),
            pltpu.VMEM((T // VL,), jnp.int32),
            pltpu.VMEM((PW // VL + VL,), jnp.int32),
            pltpu.SemaphoreType.DMA,
            pltpu.SemaphoreType.DMA,
        ],
    )


def kernel(x, duration, max_len):
    out_flat, mel2 = _regulate()(x.reshape(B * T, D),
                                 duration.astype(jnp.int32),
                                 jnp.zeros((PW,), jnp.int32))
    return out_flat.reshape(B, L, D), mel2[:, 0]
